# baseline (speedup 1.0000x reference)
import functools

import jax
import jax.numpy as jnp
from jax import lax
from jax.experimental import pallas as pl
from jax.experimental.pallas import tpu as pltpu
from jax.experimental.pallas import tpu_sc as plsc

NC = 2
NS = 16
L = 16
C = 128
NW = NC * NS


def _chunk_layout(e, kb_max):
    assert e % C == 0
    n_chunks = e // C
    full = n_chunks // NW
    extra = n_chunks - full * NW
    kb = 1
    for cand in range(kb_max, 1, -1):
        if full % cand == 0:
            kb = cand
            break
    return full, extra, kb


def _pad_rows(n):
    return -(-n // (NS * 8)) * (NS * 8)


def _mesh():
    return plsc.VectorSubcoreMesh(core_axis_name="c", subcore_axis_name="s")


def _zero_rows(zsrc, accs, row0, rpt):
    zrows = zsrc.shape[0]
    done = 0
    while done < rpt:
        nrows = min(zrows, rpt - done)
        for acc in accs:
            pltpu.sync_copy(zsrc.at[pl.ds(0, nrows)],
                            acc.at[pl.ds(row0 + done, nrows)])
        done += nrows


def _copy_row1(eiblk, dblk, kb):
    def body(t, _):
        jj = t // (C // L)
        k = t % (C // L)
        dblk[jj, pl.ds(k * L, L)] = eiblk[1, pl.ds(jj * C + k * L, L)]
        return 0
    lax.fori_loop(0, kb * (C // L), body, 0, unroll=8)


def _sc_xscatter_build(n, e, d):
    full, extra, kb = _chunk_layout(e, 13)
    nblk = full // kb
    npad = _pad_rows(n)
    rpt = npad // NS

    @functools.partial(
        pl.kernel,
        out_type=jax.ShapeDtypeStruct((NC, npad, d), jnp.float32),
        mesh=_mesh(),
        scratch_types=(
            pltpu.VMEM_SHARED((npad, d), jnp.float32),
            pltpu.VMEM((C, d), jnp.float32),
            pltpu.VMEM((C, d), jnp.float32),
            pltpu.VMEM((2, kb * C), jnp.int32),
            pltpu.VMEM((kb, C), jnp.int32),
            pltpu.VMEM((2, C), jnp.int32),
            pltpu.VMEM((1, C), jnp.int32),
            pltpu.SemaphoreType.DMA,
            pltpu.SemaphoreType.DMA,
        ),
    )
    def sc_kernel(x_hbm, ei_hbm, xp_hbm,
                  acc_x, xbuf0, xbuf1, eiblk, dblk, eix, dx1,
                  sem0, sem1):
        c = lax.axis_index("c")
        s = lax.axis_index("s")
        w = c * NS + s
        xbufs = (xbuf0, xbuf1)
        sems = (sem0, sem1)
        base_e = (full * w + jnp.minimum(w, extra)) * C

        def zrow(r, _):
            def zcol(j, _):
                xbuf0[r, pl.ds(j * L, L)] = jnp.zeros((L,), jnp.float32)
                return 0
            return lax.fori_loop(0, d // L, zcol, 0)
        lax.fori_loop(0, C, zrow, 0)
        _zero_rows(xbuf0, (acc_x,), s * rpt, rpt)
        plsc.subcore_barrier()

        def gather(jj, b):
            pltpu.async_copy(x_hbm.at[eiblk.at[0, pl.ds(jj * C, C)]],
                             xbufs[b], sems[b])

        def wait(jj, b):
            pltpu.make_async_copy(x_hbm.at[eiblk.at[0, pl.ds(jj * C, C)]],
                                  xbufs[b], sems[b]).wait()

        def scatter(jj, b):
            pltpu.sync_copy(xbufs[b], acc_x.at[dblk.at[jj]], add=True)

        def block(blk, _):
            e0 = base_e + kb * C * blk
            pltpu.sync_copy(ei_hbm.at[:, pl.ds(e0, kb * C)], eiblk)
            _copy_row1(eiblk, dblk, kb)
            gather(0, 0)

            def body(jj2, _):
                jj = 2 * jj2
                wait(jj, 0)
                @pl.when(jj + 1 < kb)
                def _():
                    gather(jj + 1, 1)
                scatter(jj, 0)
                @pl.when(jj + 1 < kb)
                def _():
                    wait(jj + 1, 1)
                    @pl.when(jj + 2 < kb)
                    def _():
                        gather(jj + 2, 0)
                    scatter(jj + 1, 1)
                return 0
            lax.fori_loop(0, (kb + 1) // 2, body, 0)
            return 0
        lax.fori_loop(0, nblk, block, 0)

        if extra:
            @pl.when(w < extra)
            def _():
                e0 = base_e + full * C
                pltpu.sync_copy(ei_hbm.at[:, pl.ds(e0, C)], eix)
                _copy_row1(eix, dx1, 1)
                pltpu.async_copy(x_hbm.at[eix.at[0]], xbuf0, sem0).wait()
                pltpu.sync_copy(xbuf0, acc_x.at[dx1.at[0]], add=True)

        plsc.subcore_barrier()
        row0 = s * rpt
        pltpu.sync_copy(acc_x.at[pl.ds(row0, rpt)], xp_hbm.at[c, pl.ds(row0, rpt)])

    return sc_kernel


def _sc_aux_build(n, e, ed, d):
    full, extra, kb = _chunk_layout(e, 6)
    nblk = full // kb
    npad = _pad_rows(n)
    rpt = npad // NS

    @functools.partial(
        pl.kernel,
        out_type=jax.ShapeDtypeStruct((NC, npad, d), jnp.float32),
        mesh=_mesh(),
        scratch_types=(
            pltpu.VMEM_SHARED((npad, d), jnp.float32),
            pltpu.VMEM((C, d), jnp.float32),
            pltpu.VMEM((C, d), jnp.float32),
            pltpu.VMEM((C * ed,), jnp.float32),
            pltpu.VMEM((C * ed,), jnp.float32),
            pltpu.VMEM((2, kb * C), jnp.int32),
            pltpu.VMEM((kb, C), jnp.int32),
            pltpu.VMEM((2, C), jnp.int32),
            pltpu.VMEM((1, C), jnp.int32),
            pltpu.SemaphoreType.DMA,
            pltpu.SemaphoreType.DMA,
            pltpu.SemaphoreType.DMA,
            pltpu.SemaphoreType.DMA,
        ),
    )
    def sc_kernel(eaflat_hbm, ei_hbm, ep_hbm,
                  acc_ec, src0, src1, pack0, pack1, eiblk, dblk, eix, dx1,
                  psem0, psem1, ssem0, ssem1):
        c = lax.axis_index("c")
        s = lax.axis_index("s")
        w = c * NS + s
        srcs = (src0, src1)
        packs = (pack0, pack1)
        psems = (psem0, psem1)
        ssems = (ssem0, ssem1)
        base_e = (full * w + jnp.minimum(w, extra)) * C

        def zrow(r, _):
            def zcol(j, _):
                src0[r, pl.ds(j * L, L)] = jnp.zeros((L,), jnp.float32)
                src1[r, pl.ds(j * L, L)] = jnp.zeros((L,), jnp.float32)
                return 0
            return lax.fori_loop(0, d // L, zcol, 0)
        lax.fori_loop(0, C, zrow, 0)
        _zero_rows(src0, (acc_ec,), s * rpt, rpt)

        def orow(r, _):
            src0[r, pl.ds(ed, L)] = jnp.ones((L,), jnp.float32)
            src1[r, pl.ds(ed, L)] = jnp.ones((L,), jnp.float32)
            return 0
        lax.fori_loop(0, C, orow, 0)
        plsc.subcore_barrier()

        def pack_start(e0, jj, b):
            pltpu.async_copy(eaflat_hbm.at[pl.ds((e0 + jj * C) * ed, C * ed)],
                             packs[b], psems[b])

        def pack_wait(e0, jj, b):
            pltpu.make_async_copy(
                eaflat_hbm.at[pl.ds((e0 + jj * C) * ed, C * ed)],
                packs[b], psems[b]).wait()

        def expand(b):
            def erow(g, _):
                srcs[b][g, pl.ds(0, ed)] = packs[b][pl.ds(g * ed, ed)]
                return 0
            lax.fori_loop(0, C, erow, 0, unroll=8)

        def scat_start(jj, b):
            pltpu.async_copy(srcs[b], acc_ec.at[dblk.at[jj]],
                             ssems[b], add=True)

        def scat_wait(b):
            pltpu.make_async_copy(srcs[b], acc_ec.at[dblk.at[0]],
                                  ssems[b]).wait()

        def block(blk, _):
            e0 = base_e + kb * C * blk
            pltpu.sync_copy(ei_hbm.at[:, pl.ds(e0, kb * C)], eiblk)
            _copy_row1(eiblk, dblk, kb)
            pack_start(e0, 0, 0)

            def body(jj2, _):
                jj = 2 * jj2
                pack_wait(e0, jj, 0)
                @pl.when(jj + 1 < kb)
                def _():
                    pack_start(e0, jj + 1, 1)
                @pl.when(jj2 > 0)
                def _():
                    scat_wait(0)
                expand(0)
                scat_start(jj, 0)
                @pl.when(jj + 1 < kb)
                def _():
                    pack_wait(e0, jj + 1, 1)
                    @pl.when(jj + 2 < kb)
                    def _():
                        pack_start(e0, jj + 2, 0)
                    @pl.when(jj2 > 0)
                    def _():
                        scat_wait(1)
                    expand(1)
                    scat_start(jj + 1, 1)
                return 0
            lax.fori_loop(0, (kb + 1) // 2, body, 0)
            scat_wait(0)
            if kb > 1:
                scat_wait(1)
            return 0
        lax.fori_loop(0, nblk, block, 0)

        if extra:
            @pl.when(w < extra)
            def _():
                e0 = base_e + full * C
                pltpu.sync_copy(ei_hbm.at[:, pl.ds(e0, C)], eix)
                _copy_row1(eix, dx1, 1)
                pltpu.sync_copy(eaflat_hbm.at[pl.ds(e0 * ed, C * ed)], pack0)
                expand(0)
                pltpu.sync_copy(src0, acc_ec.at[dx1.at[0]], add=True)

        plsc.subcore_barrier()
        row0 = s * rpt
        pltpu.sync_copy(acc_ec.at[pl.ds(row0, rpt)], ep_hbm.at[c, pl.ds(row0, rpt)])

    return sc_kernel


def _tc_dense(xparts, ecparts, x, W_edge, b_edge, W_l, b_l, W_r):
    n, d = x.shape
    ed = W_edge.shape[0]
    bn = 1000
    assert n % bn == 0

    def body(xp, ep, x_ref, we, be, wl, bl, wr, o_ref):
        sx = xp[0] + xp[1]
        ec = ep[0] + ep[1]
        se = ec[:, 0:ed]
        cnt = ec[:, ed:ed + 1]
        summed = sx + jnp.dot(se, we[...], preferred_element_type=jnp.float32)
        summed = summed + cnt * be[...]
        agg = summed / jnp.maximum(cnt, 1.0)
        o_ref[...] = (
            jnp.dot(agg, wl[...], preferred_element_type=jnp.float32)
            + bl[...]
            + jnp.dot(x_ref[...], wr[...], preferred_element_type=jnp.float32)
        )

    return pl.pallas_call(
        body,
        grid=(n // bn,),
        in_specs=[
            pl.BlockSpec((NC, bn, d), lambda i: (0, i, 0)),
            pl.BlockSpec((NC, bn, d), lambda i: (0, i, 0)),
            pl.BlockSpec((bn, d), lambda i: (i, 0)),
            pl.BlockSpec((ed, d), lambda i: (0, 0)),
            pl.BlockSpec((1, d), lambda i: (0, 0)),
            pl.BlockSpec((d, d), lambda i: (0, 0)),
            pl.BlockSpec((1, d), lambda i: (0, 0)),
            pl.BlockSpec((d, d), lambda i: (0, 0)),
        ],
        out_specs=pl.BlockSpec((bn, d), lambda i: (i, 0)),
        out_shape=jax.ShapeDtypeStruct((n, d), jnp.float32),
    )(xparts, ecparts, x, W_edge, b_edge, W_l, b_l, W_r)


def kernel(x, edge_index, edge_attr, W_edge, b_edge, W_l, b_l, W_r):
    n, d = x.shape
    e, ed = edge_attr.shape
    assert ed == L and d % L == 0
    ei = edge_index.astype(jnp.int32)
    xparts = _sc_xscatter_build(n, e, d)(x, ei)
    ecparts = _sc_aux_build(n, e, ed, d)(edge_attr.reshape(-1), ei)
    return _tc_dense(xparts, ecparts, x, W_edge,
                     b_edge.reshape(1, d), W_l, b_l.reshape(1, d), W_r)


# speedup vs baseline: 1.0591x; 1.0591x over previous
import functools

import jax
import jax.numpy as jnp
from jax import lax
from jax.experimental import pallas as pl
from jax.experimental.pallas import tpu as pltpu
from jax.experimental.pallas import tpu_sc as plsc

NC = 2
NS = 16
L = 16
C = 128
NW = NC * NS


def _chunk_layout(e, kb_max):
    assert e % C == 0
    n_chunks = e // C
    full = n_chunks // NW
    extra = n_chunks - full * NW
    kb = 1
    for cand in range(kb_max, 1, -1):
        if full % cand == 0:
            kb = cand
            break
    return full, extra, kb


def _pad_rows(n):
    return -(-n // (NS * 8)) * (NS * 8)


def _mesh():
    return plsc.VectorSubcoreMesh(core_axis_name="c", subcore_axis_name="s")


def _zero_rows(zsrc, accs, row0, rpt):
    zrows = zsrc.shape[0]
    done = 0
    while done < rpt:
        nrows = min(zrows, rpt - done)
        for acc in accs:
            pltpu.sync_copy(zsrc.at[pl.ds(0, nrows)],
                            acc.at[pl.ds(row0 + done, nrows)])
        done += nrows


def _sc_xscatter_build(n, e, d):
    full, extra, kb = _chunk_layout(e, 26)
    nblk = full // kb
    npad = _pad_rows(n)
    rpt = npad // NS

    @functools.partial(
        pl.kernel,
        out_type=jax.ShapeDtypeStruct((NC, npad, d), jnp.float32),
        mesh=_mesh(),
        scratch_types=(
            pltpu.VMEM_SHARED((npad, d), jnp.float32),
            pltpu.VMEM((C, d), jnp.float32),
            pltpu.VMEM((C, d), jnp.float32),
            pltpu.VMEM((kb * C,), jnp.int32),
            pltpu.VMEM((kb, C), jnp.int32),
            pltpu.VMEM((C,), jnp.int32),
            pltpu.VMEM((C,), jnp.int32),
            pltpu.SemaphoreType.DMA,
            pltpu.SemaphoreType.DMA,
            pltpu.SemaphoreType.DMA,
        ),
    )
    def sc_kernel(x_hbm, ei_hbm, xp_hbm,
                  acc_x, xbuf0, xbuf1, sblk, dblk, sx1, dx1,
                  sem0, sem1, isem):
        c = lax.axis_index("c")
        s = lax.axis_index("s")
        w = c * NS + s
        xbufs = (xbuf0, xbuf1)
        sems = (sem0, sem1)
        base_e = (full * w + jnp.minimum(w, extra)) * C

        def zrow(r, _):
            def zcol(j, _):
                xbuf0[r, pl.ds(j * L, L)] = jnp.zeros((L,), jnp.float32)
                return 0
            return lax.fori_loop(0, d // L, zcol, 0)
        lax.fori_loop(0, C, zrow, 0)
        _zero_rows(xbuf0, (acc_x,), s * rpt, rpt)
        plsc.subcore_barrier()

        def gather(jj, b):
            pltpu.async_copy(x_hbm.at[sblk.at[pl.ds(jj * C, C)]],
                             xbufs[b], sems[b])

        def wait(jj, b):
            pltpu.make_async_copy(x_hbm.at[sblk.at[pl.ds(jj * C, C)]],
                                  xbufs[b], sems[b]).wait()

        def scatter(jj, b):
            pltpu.sync_copy(xbufs[b], acc_x.at[dblk.at[jj]], add=True)

        def block(blk, _):
            e0 = base_e + kb * C * blk
            pltpu.sync_copy(ei_hbm.at[pl.ds(e0, kb * C)], sblk)
            for jj in range(kb):
                pltpu.async_copy(ei_hbm.at[pl.ds(e + e0 + jj * C, C)],
                                 dblk.at[jj], isem)
            for jj in range(kb):
                pltpu.make_async_copy(ei_hbm.at[pl.ds(e + e0 + jj * C, C)],
                                      dblk.at[jj], isem).wait()
            gather(0, 0)

            def body(jj2, _):
                jj = 2 * jj2
                wait(jj, 0)
                @pl.when(jj + 1 < kb)
                def _():
                    gather(jj + 1, 1)
                scatter(jj, 0)
                @pl.when(jj + 1 < kb)
                def _():
                    wait(jj + 1, 1)
                    @pl.when(jj + 2 < kb)
                    def _():
                        gather(jj + 2, 0)
                    scatter(jj + 1, 1)
                return 0
            lax.fori_loop(0, (kb + 1) // 2, body, 0)
            return 0
        lax.fori_loop(0, nblk, block, 0)

        if extra:
            @pl.when(w < extra)
            def _():
                e0 = base_e + full * C
                pltpu.sync_copy(ei_hbm.at[pl.ds(e0, C)], sx1)
                pltpu.sync_copy(ei_hbm.at[pl.ds(e + e0, C)], dx1)
                pltpu.async_copy(x_hbm.at[sx1], xbuf0, sem0).wait()
                pltpu.sync_copy(xbuf0, acc_x.at[dx1], add=True)

        plsc.subcore_barrier()
        row0 = s * rpt
        pltpu.sync_copy(acc_x.at[pl.ds(row0, rpt)], xp_hbm.at[c, pl.ds(row0, rpt)])

    return sc_kernel


def _sc_aux_build(n, e, ed, d):
    full, extra, kb = _chunk_layout(e, 26)
    nblk = full // kb
    npad = _pad_rows(n)
    rpt = npad // NS

    @functools.partial(
        pl.kernel,
        out_type=jax.ShapeDtypeStruct((NC, npad, d), jnp.float32),
        mesh=_mesh(),
        scratch_types=(
            pltpu.VMEM_SHARED((npad, d), jnp.float32),
            pltpu.VMEM((C, d), jnp.float32),
            pltpu.VMEM((C, d), jnp.float32),
            pltpu.VMEM((C * ed,), jnp.float32),
            pltpu.VMEM((C * ed,), jnp.float32),
            pltpu.VMEM((kb, C), jnp.int32),
            pltpu.VMEM((C,), jnp.int32),
            pltpu.SemaphoreType.DMA,
            pltpu.SemaphoreType.DMA,
            pltpu.SemaphoreType.DMA,
            pltpu.SemaphoreType.DMA,
            pltpu.SemaphoreType.DMA,
        ),
    )
    def sc_kernel(eaflat_hbm, ei_hbm, ep_hbm,
                  acc_ec, src0, src1, pack0, pack1, dblk, dx1,
                  psem0, psem1, ssem0, ssem1, isem):
        c = lax.axis_index("c")
        s = lax.axis_index("s")
        w = c * NS + s
        srcs = (src0, src1)
        packs = (pack0, pack1)
        psems = (psem0, psem1)
        ssems = (ssem0, ssem1)
        base_e = (full * w + jnp.minimum(w, extra)) * C

        def zrow(r, _):
            def zcol(j, _):
                src0[r, pl.ds(j * L, L)] = jnp.zeros((L,), jnp.float32)
                src1[r, pl.ds(j * L, L)] = jnp.zeros((L,), jnp.float32)
                return 0
            return lax.fori_loop(0, d // L, zcol, 0)
        lax.fori_loop(0, C, zrow, 0)
        _zero_rows(src0, (acc_ec,), s * rpt, rpt)

        def orow(r, _):
            src0[r, pl.ds(ed, L)] = jnp.ones((L,), jnp.float32)
            src1[r, pl.ds(ed, L)] = jnp.ones((L,), jnp.float32)
            return 0
        lax.fori_loop(0, C, orow, 0)
        plsc.subcore_barrier()

        def pack_start(e0, jj, b):
            pltpu.async_copy(eaflat_hbm.at[pl.ds((e0 + jj * C) * ed, C * ed)],
                             packs[b], psems[b])

        def pack_wait(e0, jj, b):
            pltpu.make_async_copy(
                eaflat_hbm.at[pl.ds((e0 + jj * C) * ed, C * ed)],
                packs[b], psems[b]).wait()

        def expand(b):
            def erow(g, _):
                srcs[b][g, pl.ds(0, ed)] = packs[b][pl.ds(g * ed, ed)]
                return 0
            lax.fori_loop(0, C, erow, 0, unroll=8)

        def scat_start(jj, b):
            pltpu.async_copy(srcs[b], acc_ec.at[dblk.at[jj]],
                             ssems[b], add=True)

        def scat_wait(b):
            pltpu.make_async_copy(srcs[b], acc_ec.at[dblk.at[0]],
                                  ssems[b]).wait()

        def block(blk, _):
            e0 = base_e + kb * C * blk
            for jj in range(kb):
                pltpu.async_copy(ei_hbm.at[pl.ds(e + e0 + jj * C, C)],
                                 dblk.at[jj], isem)
            for jj in range(kb):
                pltpu.make_async_copy(ei_hbm.at[pl.ds(e + e0 + jj * C, C)],
                                      dblk.at[jj], isem).wait()
            pack_start(e0, 0, 0)

            def body(jj2, _):
                jj = 2 * jj2
                pack_wait(e0, jj, 0)
                @pl.when(jj + 1 < kb)
                def _():
                    pack_start(e0, jj + 1, 1)
                @pl.when(jj2 > 0)
                def _():
                    scat_wait(0)
                expand(0)
                scat_start(jj, 0)
                @pl.when(jj + 1 < kb)
                def _():
                    pack_wait(e0, jj + 1, 1)
                    @pl.when(jj + 2 < kb)
                    def _():
                        pack_start(e0, jj + 2, 0)
                    @pl.when(jj2 > 0)
                    def _():
                        scat_wait(1)
                    expand(1)
                    scat_start(jj + 1, 1)
                return 0
            lax.fori_loop(0, (kb + 1) // 2, body, 0)
            scat_wait(0)
            if kb > 1:
                scat_wait(1)
            return 0
        lax.fori_loop(0, nblk, block, 0)

        if extra:
            @pl.when(w < extra)
            def _():
                e0 = base_e + full * C
                pltpu.sync_copy(ei_hbm.at[pl.ds(e + e0, C)], dx1)
                pltpu.sync_copy(eaflat_hbm.at[pl.ds(e0 * ed, C * ed)], pack0)
                expand(0)
                pltpu.sync_copy(src0, acc_ec.at[dx1], add=True)

        plsc.subcore_barrier()
        row0 = s * rpt
        pltpu.sync_copy(acc_ec.at[pl.ds(row0, rpt)], ep_hbm.at[c, pl.ds(row0, rpt)])

    return sc_kernel


def _tc_dense(xparts, ecparts, x, W_edge, b_edge, W_l, b_l, W_r):
    n, d = x.shape
    ed = W_edge.shape[0]
    bn = 1000
    assert n % bn == 0

    def body(xp, ep, x_ref, we, be, wl, bl, wr, o_ref):
        sx = xp[0] + xp[1]
        ec = ep[0] + ep[1]
        se = ec[:, 0:ed]
        cnt = ec[:, ed:ed + 1]
        summed = sx + jnp.dot(se, we[...], preferred_element_type=jnp.float32)
        summed = summed + cnt * be[...]
        agg = summed / jnp.maximum(cnt, 1.0)
        o_ref[...] = (
            jnp.dot(agg, wl[...], preferred_element_type=jnp.float32)
            + bl[...]
            + jnp.dot(x_ref[...], wr[...], preferred_element_type=jnp.float32)
        )

    return pl.pallas_call(
        body,
        grid=(n // bn,),
        in_specs=[
            pl.BlockSpec((NC, bn, d), lambda i: (0, i, 0)),
            pl.BlockSpec((NC, bn, d), lambda i: (0, i, 0)),
            pl.BlockSpec((bn, d), lambda i: (i, 0)),
            pl.BlockSpec((ed, d), lambda i: (0, 0)),
            pl.BlockSpec((1, d), lambda i: (0, 0)),
            pl.BlockSpec((d, d), lambda i: (0, 0)),
            pl.BlockSpec((1, d), lambda i: (0, 0)),
            pl.BlockSpec((d, d), lambda i: (0, 0)),
        ],
        out_specs=pl.BlockSpec((bn, d), lambda i: (i, 0)),
        out_shape=jax.ShapeDtypeStruct((n, d), jnp.float32),
    )(xparts, ecparts, x, W_edge, b_edge, W_l, b_l, W_r)


def kernel(x, edge_index, edge_attr, W_edge, b_edge, W_l, b_l, W_r):
    n, d = x.shape
    e, ed = edge_attr.shape
    assert ed == L and d % L == 0
    ei_flat = edge_index.astype(jnp.int32).reshape(-1)
    xparts = _sc_xscatter_build(n, e, d)(x, ei_flat)
    ecparts = _sc_aux_build(n, e, ed, d)(edge_attr.reshape(-1), ei_flat)
    return _tc_dense(xparts, ecparts, x, W_edge,
                     b_edge.reshape(1, d), W_l, b_l.reshape(1, d), W_r)
